# Initial kernel scaffold; baseline (speedup 1.0000x reference)
import jax
import jax.numpy as jnp
from jax.experimental import pallas as pl


def kernel(states, edge_index, Ws1, bs1, Ws2, bs2, Wr1, br1, Wr2, br2, Wrp, brp, Wi1, bi1, Wi2, bi2, Wp, bp, We1, be1, We2, be2, We3, be3):
    raise NotImplementedError("write your pallas kernel here")


# speedup vs baseline: 2.1330x; 2.1330x over previous
import functools

import jax
import jax.numpy as jnp
from jax import lax
from jax.experimental import pallas as pl
from jax.experimental.pallas import tpu as pltpu
from jax.experimental.pallas import tpu_sc as plsc

_F32 = jnp.float32
_MARGIN = 0.03
_NT = 4

_NC = 2
_NS = 16
_NW = _NC * _NS
_C = 128

_TW = 384


def _node_pre_body(states_ref, ws1, bs1, ws2, bs2, wr1, wrpa, wrpb,
                   wi1, bi1, wi2, bi2, wpa, wpb, bp,
                   tsrc_ref, tdst_ref, d_ref):
    x = states_ref[...]
    h = jnp.maximum(x @ ws1[...] + bs1[...], 0.0)
    senc = jnp.maximum(h @ ws2[...] + bs2[...], 0.0)
    p = x @ wr1[...]
    a = senc @ wrpa[...]
    b = senc @ wrpb[...]
    hi = jnp.maximum(x @ wi1[...] + bi1[...], 0.0)
    eu = jnp.maximum(hi @ wi2[...] + bi2[...], 0.0)
    ux = jnp.abs(x[:, 0:1])
    uy = jnp.abs(x[:, _NT:_NT + 1])
    u = jnp.where((ux > 1.0 - _MARGIN) | (uy > 1.0 - _MARGIN), 1.0, 0.0)
    d_ref[...] = senc @ wpa[...] + (eu * u) @ wpb[...] + bp[...]
    bn = x.shape[0]
    mcols = jnp.concatenate(
        [x[:, 0:1], x[:, _NT:_NT + 1], jnp.zeros((bn, 126), _F32)], axis=1)
    tsrc_ref[:, 0:128] = p
    tsrc_ref[:, 128:256] = a
    tsrc_ref[:, 256:_TW] = mcols
    tdst_ref[:, 0:128] = p
    tdst_ref[:, 128:256] = b
    tdst_ref[:, 256:_TW] = mcols


def _edge_body(gsrc_ref, gdst_ref, wr2, br1, br2, wrpc, brp, out_ref):
    gs = gsrc_ref[...]
    gd = gdst_ref[...]
    h1 = jnp.maximum(gs[:, 0:128] - gd[:, 0:128] + br1[...], 0.0)
    er = jnp.maximum(h1 @ wr2[...] + br2[...], 0.0)
    t = er @ wrpc[...]
    relx = gs[:, 256:257] - gd[:, 256:257]
    rely = gs[:, 257:258] - gd[:, 257:258]
    sel = jnp.where(
        (jnp.abs(relx) > 2.0 * _MARGIN) | (jnp.abs(rely) > 2.0 * _MARGIN),
        1.0, 0.0)
    out_ref[...] = jnp.maximum(
        gs[:, 128:256] + gd[:, 128:256] + t + brp[...], 0.0) * sel


def _head_body(agg_ref, d_ref, wp1, we1, be1, we2, be2, we3, be3, out_ref):
    agg = agg_ref[0] + agg_ref[1]
    ne = jnp.maximum(agg @ wp1[...] + d_ref[...], 0.0)
    hh = jnp.maximum(ne @ we1[...] + be1[...], 0.0)
    hh = jnp.maximum(hh @ we2[...] + be2[...], 0.0)
    out_ref[...] = hh @ we3[...] + be3[...]


def kernel(states, edge_index, Ws1, bs1, Ws2, bs2, Wr1, br1, Wr2, br2,
           Wrp, brp, Wi1, bi1, Wi2, bi2, Wp, bp, We1, be1, We2, be2,
           We3, be3):
    n, s_dim = states.shape
    e = edge_index.shape[1]
    ef = Wrp.shape[1]
    g_dim = We3.shape[1]

    per_w = -(-e // (_NW * _C)) * _C
    ep = per_w * _NW
    n_pad = -(-n // 128) * 128
    zr = n_pad // _NS

    src = jnp.pad(edge_index[0], (0, ep - e))
    dst = jnp.pad(edge_index[1], (0, ep - e))

    bn = 2000
    full = lambda shp: pl.BlockSpec(shp, lambda i: (0,) * len(shp))
    row = lambda w: pl.BlockSpec((1, w), lambda i: (0, 0))
    tsrc, tdst, dvec = pl.pallas_call(
        _node_pre_body,
        grid=(n // bn,),
        in_specs=[
            pl.BlockSpec((bn, s_dim), lambda i: (i, 0)),
            full((s_dim, 128)), row(128), full((128, 128)), row(128),
            full((s_dim, 128)), full((128, 128)), full((128, 128)),
            full((s_dim, 128)), row(128), full((128, 16)), row(16),
            full((128, 128)), full((16, 128)), row(128),
        ],
        out_specs=[
            pl.BlockSpec((bn, _TW), lambda i: (i, 0)),
            pl.BlockSpec((bn, _TW), lambda i: (i, 0)),
            pl.BlockSpec((bn, 128), lambda i: (i, 0)),
        ],
        out_shape=[
            jax.ShapeDtypeStruct((n, _TW), _F32),
            jax.ShapeDtypeStruct((n, _TW), _F32),
            jax.ShapeDtypeStruct((n, 128), _F32),
        ],
    )(states, Ws1, bs1.reshape(1, -1), Ws2, bs2.reshape(1, -1),
      Wr1, Wrp[0:128], Wrp[128:256],
      Wi1, bi1.reshape(1, -1), Wi2, bi2.reshape(1, -1),
      Wp[0:128], Wp[128:144], bp.reshape(1, -1))

    mesh = plsc.VectorSubcoreMesh(core_axis_name="c", subcore_axis_name="s")

    @functools.partial(
        pl.kernel,
        mesh=mesh,
        out_type=[jax.ShapeDtypeStruct((ep, _TW), _F32),
                  jax.ShapeDtypeStruct((ep, _TW), _F32)],
        scratch_types=[
            pltpu.VMEM((_C,), jnp.int32),
            pltpu.VMEM((_C,), jnp.int32),
            pltpu.VMEM((_C, _TW), _F32),
            pltpu.VMEM((_C, _TW), _F32),
            pltpu.SemaphoreType.DMA,
            pltpu.SemaphoreType.DMA,
        ],
    )
    def _gather_k(src_hbm, dst_hbm, ts_hbm, td_hbm, gs_hbm, gd_hbm,
                  idxs_v, idxd_v, rows_s, rows_d, sem_s, sem_d):
        c = lax.axis_index("c")
        s = lax.axis_index("s")
        base = (c * _NS + s) * per_w

        def body(i, carry):
            off = base + i * _C
            pltpu.sync_copy(src_hbm.at[pl.ds(off, _C)], idxs_v)
            pltpu.sync_copy(dst_hbm.at[pl.ds(off, _C)], idxd_v)
            cp1 = pltpu.async_copy(ts_hbm.at[idxs_v], rows_s, sem_s)
            cp2 = pltpu.async_copy(td_hbm.at[idxd_v], rows_d, sem_d)
            cp1.wait()
            cp2.wait()
            pltpu.sync_copy(rows_s, gs_hbm.at[pl.ds(off, _C)])
            pltpu.sync_copy(rows_d, gd_hbm.at[pl.ds(off, _C)])
            return carry

        lax.fori_loop(0, per_w // _C, body, 0)

    gsrc, gdst = _gather_k(src, dst, tsrc, tdst)

    be = 512
    eff = pl.pallas_call(
        _edge_body,
        grid=(ep // be,),
        in_specs=[
            pl.BlockSpec((be, _TW), lambda i: (i, 0)),
            pl.BlockSpec((be, _TW), lambda i: (i, 0)),
            full((128, 128)), row(128), row(128), full((128, ef)), row(ef),
        ],
        out_specs=pl.BlockSpec((be, ef), lambda i: (i, 0)),
        out_shape=jax.ShapeDtypeStruct((ep, ef), _F32),
    )(gsrc, gdst, Wr2, br1.reshape(1, -1), br2.reshape(1, -1),
      Wrp[256:384], brp.reshape(1, -1))

    zeros_blk = jnp.zeros((zr, ef), _F32)

    @functools.partial(
        pl.kernel,
        mesh=mesh,
        out_type=jax.ShapeDtypeStruct((_NC, n_pad, ef), _F32),
        scratch_types=[
            pltpu.VMEM((_C,), jnp.int32),
            pltpu.VMEM((_C, ef), _F32),
            pltpu.VMEM_SHARED((n_pad, ef), _F32),
        ],
    )
    def _scatter_k(dst_hbm, eff_hbm, z_hbm, agg_hbm, idx_v, val_v, acc_sh):
        c = lax.axis_index("c")
        s = lax.axis_index("s")
        pltpu.sync_copy(z_hbm, acc_sh.at[pl.ds(s * zr, zr)])
        plsc.subcore_barrier()
        base = (c * _NS + s) * per_w

        def body(i, carry):
            off = base + i * _C
            pltpu.sync_copy(dst_hbm.at[pl.ds(off, _C)], idx_v)
            pltpu.sync_copy(eff_hbm.at[pl.ds(off, _C)], val_v)
            pltpu.sync_copy(val_v, acc_sh.at[idx_v], add=True)
            return carry

        lax.fori_loop(0, per_w // _C, body, 0)
        plsc.subcore_barrier()
        pltpu.sync_copy(acc_sh.at[pl.ds(s * zr, zr)],
                        agg_hbm.at[c, pl.ds(s * zr, zr)])

    aggp = _scatter_k(dst, eff, zeros_blk)

    agg2 = aggp[:, :n, :]
    g_out = pl.pallas_call(
        _head_body,
        grid=(n // bn,),
        in_specs=[
            pl.BlockSpec((_NC, bn, ef), lambda i: (0, i, 0)),
            pl.BlockSpec((bn, 128), lambda i: (i, 0)),
            full((128, 128)),
            full((ef, 128)), row(128), full((128, 128)), row(128),
            full((128, g_dim)), row(g_dim),
        ],
        out_specs=pl.BlockSpec((bn, g_dim), lambda i: (i, 0)),
        out_shape=jax.ShapeDtypeStruct((n, g_dim), _F32),
    )(agg2, dvec, Wp[0:128], We1, be1.reshape(1, -1),
      We2, be2.reshape(1, -1), We3, be3.reshape(1, -1))

    return g_out


# speedup vs baseline: 2.5329x; 1.1874x over previous
import functools

import jax
import jax.numpy as jnp
from jax import lax
from jax.experimental import pallas as pl
from jax.experimental.pallas import tpu as pltpu
from jax.experimental.pallas import tpu_sc as plsc

_F32 = jnp.float32
_BF16 = jnp.bfloat16
_I32 = jnp.int32
_U32 = jnp.uint32
_MARGIN = 0.03
_NT = 4

_NC = 2
_NS = 16
_NW = _NC * _NS
_C = 128
_L = 16


def _pack_body(states_ref, ws1, bs1, ws2, bs2, wr1, wrpa, wrpb,
               wi1, bi1, wi2, bi2, wpa, wpb, bp,
               tsrc_ref, tdst_ref, d_ref):
    x = states_ref[...]
    h = jnp.maximum(x @ ws1[...] + bs1[...], 0.0)
    senc = jnp.maximum(h @ ws2[...] + bs2[...], 0.0)
    p = x @ wr1[...]
    a = senc @ wrpa[...]
    b = senc @ wrpb[...]
    hi = jnp.maximum(x @ wi1[...] + bi1[...], 0.0)
    eu = jnp.maximum(hi @ wi2[...] + bi2[...], 0.0)
    ux = jnp.abs(x[:, 0:1])
    uy = jnp.abs(x[:, _NT:_NT + 1])
    u = jnp.where((ux > 1.0 - _MARGIN) | (uy > 1.0 - _MARGIN), 1.0, 0.0)
    d_ref[...] = senc @ wpa[...] + (eu * u) @ wpb[...] + bp[...]

    def bits(v):
        return lax.bitcast_convert_type(v.astype(_BF16).astype(_F32), _U32)

    pw = bits(p) >> 16
    mask_hi = jnp.uint32(0xFFFF0000)
    tsrc_ref[...] = lax.bitcast_convert_type(pw | (bits(a) & mask_hi), _I32)
    tdst_ref[...] = lax.bitcast_convert_type(pw | (bits(b) & mask_hi), _I32)


def _edge_body(gsrc_ref, gdst_ref, wr2, br1, br2, wrpc, brp, out_ref):
    us = gsrc_ref[...]
    ud = gdst_ref[...]
    mask_hi = jnp.int32(-65536)
    ps = lax.bitcast_convert_type(us << 16, _F32)
    pd = lax.bitcast_convert_type(ud << 16, _F32)
    asrc = lax.bitcast_convert_type(us & mask_hi, _F32)
    bdst = lax.bitcast_convert_type(ud & mask_hi, _F32)
    h1 = jnp.maximum(ps - pd + br1[...], 0.0)
    er = jnp.maximum(h1 @ wr2[...] + br2[...], 0.0)
    t = er @ wrpc[...]
    out_ref[...] = jnp.maximum(asrc + bdst + t + brp[...], 0.0)


def _head_body(agg_ref, d_ref, wp1, we1, be1, we2, be2, we3, be3, out_ref):
    agg = agg_ref[0] + agg_ref[1]
    ne = jnp.maximum(agg @ wp1[...] + d_ref[...], 0.0)
    hh = jnp.maximum(ne @ we1[...] + be1[...], 0.0)
    hh = jnp.maximum(hh @ we2[...] + be2[...], 0.0)
    out_ref[...] = hh @ we3[...] + be3[...]


def kernel(states, edge_index, Ws1, bs1, Ws2, bs2, Wr1, br1, Wr2, br2,
           Wrp, brp, Wi1, bi1, Wi2, bi2, Wp, bp, We1, be1, We2, be2,
           We3, be3):
    n, s_dim = states.shape
    e = edge_index.shape[1]
    ef = Wrp.shape[1]
    g_dim = We3.shape[1]

    per_w = -(-e // (_NW * 2 * _C)) * 2 * _C
    ep = per_w * _NW
    chunks = per_w // _C
    n_pad = -(-n // 128) * 128
    zr = n_pad // _NS
    dummy = n_pad - 1

    src3 = jnp.pad(edge_index[0], (0, ep - e)).reshape(_NW, chunks, _C)
    dst3 = jnp.pad(edge_index[1], (0, ep - e)).reshape(_NW, chunks, _C)
    s0 = states[:, 0]
    s4 = states[:, _NT]

    bn = 2000
    full = lambda shp: pl.BlockSpec(shp, lambda i: (0,) * len(shp))
    row = lambda w: pl.BlockSpec((1, w), lambda i: (0, 0))
    tsrc, tdst, dvec = pl.pallas_call(
        _pack_body,
        grid=(n // bn,),
        in_specs=[
            pl.BlockSpec((bn, s_dim), lambda i: (i, 0)),
            full((s_dim, 128)), row(128), full((128, 128)), row(128),
            full((s_dim, 128)), full((128, 128)), full((128, 128)),
            full((s_dim, 128)), row(128), full((128, 16)), row(16),
            full((128, 128)), full((16, 128)), row(128),
        ],
        out_specs=[
            pl.BlockSpec((bn, 128), lambda i: (i, 0)),
            pl.BlockSpec((bn, 128), lambda i: (i, 0)),
            pl.BlockSpec((bn, 128), lambda i: (i, 0)),
        ],
        out_shape=[
            jax.ShapeDtypeStruct((n, 128), _I32),
            jax.ShapeDtypeStruct((n, 128), _I32),
            jax.ShapeDtypeStruct((n, 128), _F32),
        ],
    )(states, Ws1, bs1.reshape(1, -1), Ws2, bs2.reshape(1, -1),
      Wr1, Wrp[0:128], Wrp[128:256],
      Wi1, bi1.reshape(1, -1), Wi2, bi2.reshape(1, -1),
      Wp[0:128], Wp[128:144], bp.reshape(1, -1))

    mesh = plsc.VectorSubcoreMesh(core_axis_name="c", subcore_axis_name="s")

    @functools.partial(
        pl.kernel,
        mesh=mesh,
        out_type=[jax.ShapeDtypeStruct((ep, 128), _I32),
                  jax.ShapeDtypeStruct((ep, 128), _I32)],
        scratch_types=[
            pltpu.VMEM((chunks, _C), _I32),
            pltpu.VMEM((chunks, _C), _I32),
            pltpu.VMEM((2, _C, 128), _I32),
            pltpu.VMEM((2, _C, 128), _I32),
            pltpu.SemaphoreType.DMA,
            pltpu.SemaphoreType.DMA,
            pltpu.SemaphoreType.DMA,
        ],
    )
    def _gather_k(src_hbm, dst_hbm, ts_hbm, td_hbm, gs_hbm, gd_hbm,
                  idxs_v, idxd_v, rows_s, rows_d, sem_g, sem_o0, sem_o1):
        c = lax.axis_index("c")
        s = lax.axis_index("s")
        wid = c * _NS + s
        base = wid * per_w
        pltpu.sync_copy(src_hbm.at[wid], idxs_v)
        pltpu.sync_copy(dst_hbm.at[wid], idxd_v)
        sem_o = (sem_o0, sem_o1)

        def body(i2, carry):
            for b in range(2):
                i = i2 * 2 + b
                off = base + i * _C
                cp1 = pltpu.async_copy(ts_hbm.at[idxs_v.at[i]],
                                       rows_s.at[b], sem_g)
                cp2 = pltpu.async_copy(td_hbm.at[idxd_v.at[i]],
                                       rows_d.at[b], sem_g)
                cp1.wait()
                cp2.wait()

                @pl.when(i2 > 0)
                def _():
                    pltpu.make_async_copy(
                        rows_s.at[b], gs_hbm.at[pl.ds(0, _C)], sem_o[b]).wait()
                    pltpu.make_async_copy(
                        rows_d.at[b], gd_hbm.at[pl.ds(0, _C)], sem_o[b]).wait()

                pltpu.async_copy(rows_s.at[b], gs_hbm.at[pl.ds(off, _C)],
                                 sem_o[b])
                pltpu.async_copy(rows_d.at[b], gd_hbm.at[pl.ds(off, _C)],
                                 sem_o[b])
            return carry

        lax.fori_loop(0, chunks // 2, body, 0)
        for b in range(2):
            pltpu.make_async_copy(
                rows_s.at[b], gs_hbm.at[pl.ds(0, _C)], sem_o[b]).wait()
            pltpu.make_async_copy(
                rows_d.at[b], gd_hbm.at[pl.ds(0, _C)], sem_o[b]).wait()

    gsrc, gdst = _gather_k(src3, dst3, tsrc, tdst)

    be = 512
    eff = pl.pallas_call(
        _edge_body,
        grid=(ep // be,),
        in_specs=[
            pl.BlockSpec((be, 128), lambda i: (i, 0)),
            pl.BlockSpec((be, 128), lambda i: (i, 0)),
            full((128, 128)), row(128), row(128), full((128, ef)), row(ef),
        ],
        out_specs=pl.BlockSpec((be, ef), lambda i: (i, 0)),
        out_shape=jax.ShapeDtypeStruct((ep, ef), _F32),
    )(gsrc, gdst, Wr2, br1.reshape(1, -1), br2.reshape(1, -1),
      Wrp[256:384], brp.reshape(1, -1))

    zeros_blk = jnp.zeros((zr, ef), _F32)
    thr = jnp.float32(2.0 * _MARGIN)

    @functools.partial(
        pl.kernel,
        mesh=mesh,
        out_type=jax.ShapeDtypeStruct((_NC, n_pad, ef), _F32),
        scratch_types=[
            pltpu.VMEM((_C,), _I32),
            pltpu.VMEM((_C,), _I32),
            pltpu.VMEM((_C,), _I32),
            pltpu.VMEM((2, _C, ef), _F32),
            pltpu.VMEM((_C,), _F32),
            pltpu.VMEM((_C,), _F32),
            pltpu.VMEM((_C,), _F32),
            pltpu.VMEM((_C,), _F32),
            pltpu.VMEM_SHARED((n_pad, ef), _F32),
            pltpu.SemaphoreType.DMA,
            pltpu.SemaphoreType.DMA,
            pltpu.SemaphoreType.DMA,
        ],
    )
    def _scatter_k(src_hbm, dst_hbm, eff_hbm, s0_hbm, s4_hbm, z_hbm, agg_hbm,
                   idxs_v, idxd_v, idxm_v, val_v, s0s_v, s0d_v, s4s_v, s4d_v,
                   acc_sh, sem_v0, sem_v1, sem_m):
        c = lax.axis_index("c")
        s = lax.axis_index("s")
        wid = c * _NS + s
        base = wid * per_w
        sem_v = (sem_v0, sem_v1)
        pltpu.sync_copy(z_hbm, acc_sh.at[pl.ds(s * zr, zr)])
        plsc.subcore_barrier()
        pltpu.async_copy(eff_hbm.at[pl.ds(base, _C)], val_v.at[0], sem_v[0])

        def body(i2, carry):
            for b in range(2):
                i = i2 * 2 + b
                pltpu.sync_copy(src_hbm.at[wid, i], idxs_v)
                pltpu.sync_copy(dst_hbm.at[wid, i], idxd_v)
                g1 = pltpu.async_copy(s0_hbm.at[idxs_v], s0s_v, sem_m)
                g2 = pltpu.async_copy(s0_hbm.at[idxd_v], s0d_v, sem_m)
                g3 = pltpu.async_copy(s4_hbm.at[idxs_v], s4s_v, sem_m)
                g4 = pltpu.async_copy(s4_hbm.at[idxd_v], s4d_v, sem_m)
                pltpu.make_async_copy(
                    eff_hbm.at[pl.ds(0, _C)], val_v.at[b], sem_v[b]).wait()

                @pl.when(i2 * 2 + b + 1 < chunks)
                def _():
                    off = base + (i + 1) * _C
                    pltpu.async_copy(eff_hbm.at[pl.ds(off, _C)],
                                     val_v.at[1 - b], sem_v[1 - b])

                g1.wait()
                g2.wait()
                g3.wait()
                g4.wait()
                for j in range(_C // _L):
                    sl = pl.ds(j * _L, _L)
                    di = idxd_v[sl]
                    relx = s0s_v[sl] - s0d_v[sl]
                    rely = s4s_v[sl] - s4d_v[sl]
                    sel = (jnp.abs(relx) > thr) | (jnp.abs(rely) > thr)
                    idxm_v[sl] = jnp.where(sel, di, dummy)

                pltpu.sync_copy(val_v.at[b], acc_sh.at[idxm_v], add=True)
            return carry

        lax.fori_loop(0, chunks // 2, body, 0)
        plsc.subcore_barrier()
        pltpu.sync_copy(acc_sh.at[pl.ds(s * zr, zr)],
                        agg_hbm.at[c, pl.ds(s * zr, zr)])

    aggp = _scatter_k(src3, dst3, eff, s0, s4, zeros_blk)

    agg2 = aggp[:, :n, :]
    g_out = pl.pallas_call(
        _head_body,
        grid=(n // bn,),
        in_specs=[
            pl.BlockSpec((_NC, bn, ef), lambda i: (0, i, 0)),
            pl.BlockSpec((bn, 128), lambda i: (i, 0)),
            full((128, 128)),
            full((ef, 128)), row(128), full((128, 128)), row(128),
            full((128, g_dim)), row(g_dim),
        ],
        out_specs=pl.BlockSpec((bn, g_dim), lambda i: (i, 0)),
        out_shape=jax.ShapeDtypeStruct((n, g_dim), _F32),
    )(agg2, dvec, Wp[0:128], We1, be1.reshape(1, -1),
      We2, be2.reshape(1, -1), We3, be3.reshape(1, -1))

    return g_out


# speedup vs baseline: 3.2563x; 1.2856x over previous
import functools

import jax
import jax.numpy as jnp
from jax import lax
from jax.experimental import pallas as pl
from jax.experimental.pallas import tpu as pltpu
from jax.experimental.pallas import tpu_sc as plsc

_F32 = jnp.float32
_BF16 = jnp.bfloat16
_I32 = jnp.int32
_U32 = jnp.uint32
_MARGIN = 0.03
_NT = 4

_NC = 2
_NS = 16
_NW = _NC * _NS
_C = 128
_L = 16


def _pack_body(states_ref, ws1, bs1, ws2, bs2, wr1, wrpa, wrpb,
               wi1, bi1, wi2, bi2, wpa, wpb, bp,
               tsrc_ref, tdst_ref, d_ref):
    x = states_ref[...]
    h = jnp.maximum(x @ ws1[...] + bs1[...], 0.0)
    senc = jnp.maximum(h @ ws2[...] + bs2[...], 0.0)
    p = x @ wr1[...]
    a = senc @ wrpa[...]
    b = senc @ wrpb[...]
    hi = jnp.maximum(x @ wi1[...] + bi1[...], 0.0)
    eu = jnp.maximum(hi @ wi2[...] + bi2[...], 0.0)
    ux = jnp.abs(x[:, 0:1])
    uy = jnp.abs(x[:, _NT:_NT + 1])
    u = jnp.where((ux > 1.0 - _MARGIN) | (uy > 1.0 - _MARGIN), 1.0, 0.0)
    d_ref[...] = senc @ wpa[...] + (eu * u) @ wpb[...] + bp[...]

    def bits(v):
        return lax.bitcast_convert_type(v.astype(_BF16).astype(_F32), _U32)

    pw = bits(p) >> 16
    mask_hi = jnp.uint32(0xFFFF0000)
    tsrc_ref[...] = lax.bitcast_convert_type(pw | (bits(a) & mask_hi), _I32)
    tdst_ref[...] = lax.bitcast_convert_type(pw | (bits(b) & mask_hi), _I32)


def _edge_body(gsrc_ref, gdst_ref, wr2, br1, br2, wrpc, brp, out_ref):
    us = gsrc_ref[...]
    ud = gdst_ref[...]
    mask_hi = jnp.int32(-65536)
    ps = lax.bitcast_convert_type(us << 16, _F32)
    pd = lax.bitcast_convert_type(ud << 16, _F32)
    asrc = lax.bitcast_convert_type(us & mask_hi, _F32)
    bdst = lax.bitcast_convert_type(ud & mask_hi, _F32)
    h1 = jnp.maximum(ps - pd + br1[...], 0.0).astype(_BF16)
    er = lax.dot_general(h1, wr2[...], (((1,), (0,)), ((), ())),
                         preferred_element_type=_F32) + br2[...]
    er = jnp.maximum(er, 0.0).astype(_BF16)
    t = lax.dot_general(er, wrpc[...], (((1,), (0,)), ((), ())),
                        preferred_element_type=_F32)
    out_ref[...] = jnp.maximum(asrc + bdst + t + brp[...], 0.0)


def _head_body(agg_ref, d_ref, wp1, we1, be1, we2, be2, we3, be3, out_ref):
    agg = agg_ref[0] + agg_ref[1]
    ne = jnp.maximum(agg @ wp1[...] + d_ref[...], 0.0)
    hh = jnp.maximum(ne @ we1[...] + be1[...], 0.0)
    hh = jnp.maximum(hh @ we2[...] + be2[...], 0.0)
    out_ref[...] = hh @ we3[...] + be3[...]


def kernel(states, edge_index, Ws1, bs1, Ws2, bs2, Wr1, br1, Wr2, br2,
           Wrp, brp, Wi1, bi1, Wi2, bi2, Wp, bp, We1, be1, We2, be2,
           We3, be3):
    n, s_dim = states.shape
    e = edge_index.shape[1]
    ef = Wrp.shape[1]
    g_dim = We3.shape[1]

    per_w = -(-e // (_NW * 2 * _C)) * 2 * _C
    ep = per_w * _NW
    chunks = per_w // _C
    n_pad = -(-n // 128) * 128
    zr = n_pad // _NS
    dummy = n_pad - 1

    src3 = jnp.pad(edge_index[0], (0, ep - e)).reshape(_NW, chunks, _C)
    dst3 = jnp.pad(edge_index[1], (0, ep - e)).reshape(_NW, chunks, _C)
    s0 = states[:, 0]
    s4 = states[:, _NT]

    bn = 2000
    full = lambda shp: pl.BlockSpec(shp, lambda i: (0,) * len(shp))
    row = lambda w: pl.BlockSpec((1, w), lambda i: (0, 0))
    tsrc, tdst, dvec = pl.pallas_call(
        _pack_body,
        grid=(n // bn,),
        in_specs=[
            pl.BlockSpec((bn, s_dim), lambda i: (i, 0)),
            full((s_dim, 128)), row(128), full((128, 128)), row(128),
            full((s_dim, 128)), full((128, 128)), full((128, 128)),
            full((s_dim, 128)), row(128), full((128, 16)), row(16),
            full((128, 128)), full((16, 128)), row(128),
        ],
        out_specs=[
            pl.BlockSpec((bn, 128), lambda i: (i, 0)),
            pl.BlockSpec((bn, 128), lambda i: (i, 0)),
            pl.BlockSpec((bn, 128), lambda i: (i, 0)),
        ],
        out_shape=[
            jax.ShapeDtypeStruct((n, 128), _I32),
            jax.ShapeDtypeStruct((n, 128), _I32),
            jax.ShapeDtypeStruct((n, 128), _F32),
        ],
    )(states, Ws1, bs1.reshape(1, -1), Ws2, bs2.reshape(1, -1),
      Wr1, Wrp[0:128], Wrp[128:256],
      Wi1, bi1.reshape(1, -1), Wi2, bi2.reshape(1, -1),
      Wp[0:128], Wp[128:144], bp.reshape(1, -1))

    mesh = plsc.VectorSubcoreMesh(core_axis_name="c", subcore_axis_name="s")

    @functools.partial(
        pl.kernel,
        mesh=mesh,
        out_type=[jax.ShapeDtypeStruct((ep, 128), _I32),
                  jax.ShapeDtypeStruct((ep, 128), _I32)],
        scratch_types=[
            pltpu.VMEM((chunks, _C), _I32),
            pltpu.VMEM((chunks, _C), _I32),
            pltpu.VMEM((2, _C, 128), _I32),
            pltpu.VMEM((2, _C, 128), _I32),
            pltpu.SemaphoreType.DMA,
            pltpu.SemaphoreType.DMA,
            pltpu.SemaphoreType.DMA,
            pltpu.SemaphoreType.DMA,
        ],
    )
    def _gather_k(src_hbm, dst_hbm, ts_hbm, td_hbm, gs_hbm, gd_hbm,
                  idxs_v, idxd_v, rows_s, rows_d, sem_g0, sem_g1,
                  sem_o0, sem_o1):
        c = lax.axis_index("c")
        s = lax.axis_index("s")
        wid = s * _NC + c
        base = wid * per_w
        pltpu.sync_copy(src_hbm.at[wid], idxs_v)
        pltpu.sync_copy(dst_hbm.at[wid], idxd_v)
        sem_g = (sem_g0, sem_g1)
        sem_o = (sem_o0, sem_o1)

        def gathers(i, b):
            pltpu.async_copy(ts_hbm.at[idxs_v.at[i]], rows_s.at[b], sem_g[b])
            pltpu.async_copy(td_hbm.at[idxd_v.at[i]], rows_d.at[b], sem_g[b])

        def wait_gathers(b):
            pltpu.make_async_copy(
                ts_hbm.at[idxs_v.at[0]], rows_s.at[b], sem_g[b]).wait()
            pltpu.make_async_copy(
                td_hbm.at[idxd_v.at[0]], rows_d.at[b], sem_g[b]).wait()

        def wait_outs(b):
            pltpu.make_async_copy(
                rows_s.at[b], gs_hbm.at[pl.ds(0, _C)], sem_o[b]).wait()
            pltpu.make_async_copy(
                rows_d.at[b], gd_hbm.at[pl.ds(0, _C)], sem_o[b]).wait()

        gathers(0, 0)

        def body(i2, carry):
            for b in range(2):
                i = i2 * 2 + b
                nb = 1 - b

                if b == 0:
                    @pl.when(i2 > 0)
                    def _():
                        wait_outs(nb)
                else:
                    wait_outs(nb)

                @pl.when(i + 1 < chunks)
                def _():
                    gathers(i + 1, nb)

                wait_gathers(b)
                off = base + i * _C
                pltpu.async_copy(rows_s.at[b], gs_hbm.at[pl.ds(off, _C)],
                                 sem_o[b])
                pltpu.async_copy(rows_d.at[b], gd_hbm.at[pl.ds(off, _C)],
                                 sem_o[b])
            return carry

        lax.fori_loop(0, chunks // 2, body, 0)
        wait_outs(1)

    gsrc, gdst = _gather_k(src3, dst3, tsrc, tdst)

    be = 1024
    eff = pl.pallas_call(
        _edge_body,
        grid=(ep // be,),
        in_specs=[
            pl.BlockSpec((be, 128), lambda i: (i, 0)),
            pl.BlockSpec((be, 128), lambda i: (i, 0)),
            full((128, 128)), row(128), row(128), full((128, ef)), row(ef),
        ],
        out_specs=pl.BlockSpec((be, ef), lambda i: (i, 0)),
        out_shape=jax.ShapeDtypeStruct((ep, ef), _F32),
    )(gsrc, gdst, Wr2.astype(_BF16), br1.reshape(1, -1), br2.reshape(1, -1),
      Wrp[256:384].astype(_BF16), brp.reshape(1, -1))

    zeros_blk = jnp.zeros((zr, ef), _F32)
    thr = jnp.float32(2.0 * _MARGIN)

    @functools.partial(
        pl.kernel,
        mesh=mesh,
        out_type=jax.ShapeDtypeStruct((_NC, n_pad, ef), _F32),
        scratch_types=[
            pltpu.VMEM((_C,), _I32),
            pltpu.VMEM((_C,), _I32),
            pltpu.VMEM((2, _C), _I32),
            pltpu.VMEM((2, _C, ef), _F32),
            pltpu.VMEM((_C,), _F32),
            pltpu.VMEM((_C,), _F32),
            pltpu.VMEM((_C,), _F32),
            pltpu.VMEM((_C,), _F32),
            pltpu.VMEM_SHARED((n_pad, ef), _F32),
            pltpu.SemaphoreType.DMA,
            pltpu.SemaphoreType.DMA,
            pltpu.SemaphoreType.DMA,
            pltpu.SemaphoreType.DMA,
            pltpu.SemaphoreType.DMA,
        ],
    )
    def _scatter_k(src_hbm, dst_hbm, eff_hbm, s0_hbm, s4_hbm, z_hbm, agg_hbm,
                   idxs_v, idxd_v, idxm_v, val_v, s0s_v, s0d_v, s4s_v, s4d_v,
                   acc_sh, sem_v0, sem_v1, sem_m, sem_s0, sem_s1):
        c = lax.axis_index("c")
        s = lax.axis_index("s")
        wid = s * _NC + c
        base = wid * per_w
        sem_v = (sem_v0, sem_v1)
        sem_s = (sem_s0, sem_s1)
        pltpu.sync_copy(z_hbm, acc_sh.at[pl.ds(s * zr, zr)])
        plsc.subcore_barrier()
        pltpu.async_copy(eff_hbm.at[pl.ds(base, _C)], val_v.at[0], sem_v[0])

        def body(i2, carry):
            for b in range(2):
                i = i2 * 2 + b
                nb = 1 - b
                pltpu.sync_copy(src_hbm.at[wid, i], idxs_v)
                pltpu.sync_copy(dst_hbm.at[wid, i], idxd_v)
                g1 = pltpu.async_copy(s0_hbm.at[idxs_v], s0s_v, sem_m)
                g2 = pltpu.async_copy(s0_hbm.at[idxd_v], s0d_v, sem_m)
                g3 = pltpu.async_copy(s4_hbm.at[idxs_v], s4s_v, sem_m)
                g4 = pltpu.async_copy(s4_hbm.at[idxd_v], s4d_v, sem_m)

                @pl.when((i2 > 0) | (b > 0))
                def _():
                    pltpu.make_async_copy(
                        val_v.at[nb], acc_sh.at[idxm_v.at[nb]],
                        sem_s[nb]).wait()

                @pl.when(i + 1 < chunks)
                def _():
                    off = base + (i + 1) * _C
                    pltpu.async_copy(eff_hbm.at[pl.ds(off, _C)],
                                     val_v.at[nb], sem_v[nb])

                pltpu.make_async_copy(
                    eff_hbm.at[pl.ds(0, _C)], val_v.at[b], sem_v[b]).wait()
                g1.wait()
                g2.wait()
                g3.wait()
                g4.wait()
                for j in range(_C // _L):
                    sl = pl.ds(j * _L, _L)
                    di = idxd_v[sl]
                    relx = s0s_v[sl] - s0d_v[sl]
                    rely = s4s_v[sl] - s4d_v[sl]
                    sel = (jnp.abs(relx) > thr) | (jnp.abs(rely) > thr)
                    idxm_v[b, sl] = jnp.where(sel, di, dummy)

                pltpu.async_copy(val_v.at[b], acc_sh.at[idxm_v.at[b]],
                                 sem_s[b], add=True)
            return carry

        lax.fori_loop(0, chunks // 2, body, 0)
        pltpu.make_async_copy(
            val_v.at[1], acc_sh.at[idxm_v.at[1]], sem_s[1]).wait()
        plsc.subcore_barrier()
        pltpu.sync_copy(acc_sh.at[pl.ds(s * zr, zr)],
                        agg_hbm.at[c, pl.ds(s * zr, zr)])

    aggp = _scatter_k(src3, dst3, eff, s0, s4, zeros_blk)

    agg2 = aggp[:, :n, :]
    g_out = pl.pallas_call(
        _head_body,
        grid=(n // bn,),
        in_specs=[
            pl.BlockSpec((_NC, bn, ef), lambda i: (0, i, 0)),
            pl.BlockSpec((bn, 128), lambda i: (i, 0)),
            full((128, 128)),
            full((ef, 128)), row(128), full((128, 128)), row(128),
            full((128, g_dim)), row(g_dim),
        ],
        out_specs=pl.BlockSpec((bn, g_dim), lambda i: (i, 0)),
        out_shape=jax.ShapeDtypeStruct((n, g_dim), _F32),
    )(agg2, dvec, Wp[0:128], We1, be1.reshape(1, -1),
      We2, be2.reshape(1, -1), We3, be3.reshape(1, -1))

    return g_out


# speedup vs baseline: 3.2566x; 1.0001x over previous
import functools

import jax
import jax.numpy as jnp
from jax import lax
from jax.experimental import pallas as pl
from jax.experimental.pallas import tpu as pltpu
from jax.experimental.pallas import tpu_sc as plsc

_F32 = jnp.float32
_BF16 = jnp.bfloat16
_I32 = jnp.int32
_U32 = jnp.uint32
_MARGIN = 0.03
_NT = 4

_NC = 2
_NS = 16
_NW = _NC * _NS
_C = 128
_L = 16


def _pack_body(states_ref, ws1, bs1, ws2, bs2, wr1, wrpa, wrpb,
               wi1, bi1, wi2, bi2, wpa, wpb, bp,
               tsrc_ref, tdst_ref, d_ref):
    x = states_ref[...]
    h = jnp.maximum(x @ ws1[...] + bs1[...], 0.0)
    senc = jnp.maximum(h @ ws2[...] + bs2[...], 0.0)
    p = x @ wr1[...]
    a = senc @ wrpa[...]
    b = senc @ wrpb[...]
    hi = jnp.maximum(x @ wi1[...] + bi1[...], 0.0)
    eu = jnp.maximum(hi @ wi2[...] + bi2[...], 0.0)
    ux = jnp.abs(x[:, 0:1])
    uy = jnp.abs(x[:, _NT:_NT + 1])
    u = jnp.where((ux > 1.0 - _MARGIN) | (uy > 1.0 - _MARGIN), 1.0, 0.0)
    d_ref[...] = senc @ wpa[...] + (eu * u) @ wpb[...] + bp[...]

    def bits(v):
        return lax.bitcast_convert_type(v.astype(_BF16).astype(_F32), _U32)

    pw = bits(p) >> 16
    mask_hi = jnp.uint32(0xFFFF0000)
    tsrc_ref[...] = lax.bitcast_convert_type(pw | (bits(a) & mask_hi), _I32)
    tdst_ref[...] = lax.bitcast_convert_type(pw | (bits(b) & mask_hi), _I32)


def _edge_body(gsrc_ref, gdst_ref, wr2, br1, br2, wrpc, brp, out_ref):
    us = gsrc_ref[...]
    ud = gdst_ref[...]
    mask_hi = jnp.int32(-65536)
    ps = lax.bitcast_convert_type(us << 16, _F32)
    pd = lax.bitcast_convert_type(ud << 16, _F32)
    asrc = lax.bitcast_convert_type(us & mask_hi, _F32)
    bdst = lax.bitcast_convert_type(ud & mask_hi, _F32)
    h1 = jnp.maximum(ps - pd + br1[...], 0.0).astype(_BF16)
    er = lax.dot_general(h1, wr2[...], (((1,), (0,)), ((), ())),
                         preferred_element_type=_F32) + br2[...]
    er = jnp.maximum(er, 0.0).astype(_BF16)
    t = lax.dot_general(er, wrpc[...], (((1,), (0,)), ((), ())),
                        preferred_element_type=_F32)
    out_ref[...] = jnp.maximum(asrc + bdst + t + brp[...], 0.0)


def _head_body(agg_ref, d_ref, wp1, we1, be1, we2, be2, we3, be3, out_ref):
    agg = agg_ref[0] + agg_ref[1]
    ne = jnp.maximum(agg @ wp1[...] + d_ref[...], 0.0)
    hh = jnp.maximum(ne @ we1[...] + be1[...], 0.0)
    hh = jnp.maximum(hh @ we2[...] + be2[...], 0.0)
    out_ref[...] = hh @ we3[...] + be3[...]


def kernel(states, edge_index, Ws1, bs1, Ws2, bs2, Wr1, br1, Wr2, br2,
           Wrp, brp, Wi1, bi1, Wi2, bi2, Wp, bp, We1, be1, We2, be2,
           We3, be3):
    n, s_dim = states.shape
    e = edge_index.shape[1]
    ef = Wrp.shape[1]
    g_dim = We3.shape[1]

    per_w = -(-e // (_NW * 2 * _C)) * 2 * _C
    ep = per_w * _NW
    chunks = per_w // _C
    n_pad = -(-n // 128) * 128
    zr = n_pad // _NS
    dummy = n_pad - 1

    src3 = jnp.pad(edge_index[0], (0, ep - e)).reshape(_NW, chunks, _C)
    dst3 = jnp.pad(edge_index[1], (0, ep - e)).reshape(_NW, chunks, _C)
    c2 = 64
    pw2 = ep // _NS
    chunks2 = pw2 // c2
    src4 = src3.reshape(_NS, chunks2, c2)
    dst4 = dst3.reshape(_NS, chunks2, c2)
    s0 = states[:, 0]
    s4 = states[:, _NT]

    bn = 2000
    full = lambda shp: pl.BlockSpec(shp, lambda i: (0,) * len(shp))
    row = lambda w: pl.BlockSpec((1, w), lambda i: (0, 0))
    tsrc, tdst, dvec = pl.pallas_call(
        _pack_body,
        grid=(n // bn,),
        in_specs=[
            pl.BlockSpec((bn, s_dim), lambda i: (i, 0)),
            full((s_dim, 128)), row(128), full((128, 128)), row(128),
            full((s_dim, 128)), full((128, 128)), full((128, 128)),
            full((s_dim, 128)), row(128), full((128, 16)), row(16),
            full((128, 128)), full((16, 128)), row(128),
        ],
        out_specs=[
            pl.BlockSpec((bn, 128), lambda i: (i, 0)),
            pl.BlockSpec((bn, 128), lambda i: (i, 0)),
            pl.BlockSpec((bn, 128), lambda i: (i, 0)),
        ],
        out_shape=[
            jax.ShapeDtypeStruct((n, 128), _I32),
            jax.ShapeDtypeStruct((n, 128), _I32),
            jax.ShapeDtypeStruct((n, 128), _F32),
        ],
    )(states, Ws1, bs1.reshape(1, -1), Ws2, bs2.reshape(1, -1),
      Wr1, Wrp[0:128], Wrp[128:256],
      Wi1, bi1.reshape(1, -1), Wi2, bi2.reshape(1, -1),
      Wp[0:128], Wp[128:144], bp.reshape(1, -1))

    mesh = plsc.VectorSubcoreMesh(core_axis_name="c", subcore_axis_name="s")

    @functools.partial(
        pl.kernel,
        mesh=mesh,
        out_type=[jax.ShapeDtypeStruct((ep, 128), _I32),
                  jax.ShapeDtypeStruct((ep, 128), _I32)],
        scratch_types=[
            pltpu.VMEM((chunks, _C), _I32),
            pltpu.VMEM((chunks, _C), _I32),
            pltpu.VMEM((2, _C, 128), _I32),
            pltpu.VMEM((2, _C, 128), _I32),
            pltpu.SemaphoreType.DMA,
            pltpu.SemaphoreType.DMA,
            pltpu.SemaphoreType.DMA,
            pltpu.SemaphoreType.DMA,
        ],
    )
    def _gather_k(src_hbm, dst_hbm, ts_hbm, td_hbm, gs_hbm, gd_hbm,
                  idxs_v, idxd_v, rows_s, rows_d, sem_g0, sem_g1,
                  sem_o0, sem_o1):
        c = lax.axis_index("c")
        s = lax.axis_index("s")
        wid = s * _NC + c
        base = wid * per_w
        pltpu.sync_copy(src_hbm.at[wid], idxs_v)
        pltpu.sync_copy(dst_hbm.at[wid], idxd_v)
        sem_g = (sem_g0, sem_g1)
        sem_o = (sem_o0, sem_o1)

        def gathers(i, b):
            pltpu.async_copy(ts_hbm.at[idxs_v.at[i]], rows_s.at[b], sem_g[b])
            pltpu.async_copy(td_hbm.at[idxd_v.at[i]], rows_d.at[b], sem_g[b])

        def wait_gathers(b):
            pltpu.make_async_copy(
                ts_hbm.at[idxs_v.at[0]], rows_s.at[b], sem_g[b]).wait()
            pltpu.make_async_copy(
                td_hbm.at[idxd_v.at[0]], rows_d.at[b], sem_g[b]).wait()

        def wait_outs(b):
            pltpu.make_async_copy(
                rows_s.at[b], gs_hbm.at[pl.ds(0, _C)], sem_o[b]).wait()
            pltpu.make_async_copy(
                rows_d.at[b], gd_hbm.at[pl.ds(0, _C)], sem_o[b]).wait()

        gathers(0, 0)

        def body(i2, carry):
            for b in range(2):
                i = i2 * 2 + b
                nb = 1 - b

                if b == 0:
                    @pl.when(i2 > 0)
                    def _():
                        wait_outs(nb)
                else:
                    wait_outs(nb)

                @pl.when(i + 1 < chunks)
                def _():
                    gathers(i + 1, nb)

                wait_gathers(b)
                off = base + i * _C
                pltpu.async_copy(rows_s.at[b], gs_hbm.at[pl.ds(off, _C)],
                                 sem_o[b])
                pltpu.async_copy(rows_d.at[b], gd_hbm.at[pl.ds(off, _C)],
                                 sem_o[b])
            return carry

        lax.fori_loop(0, chunks // 2, body, 0)
        wait_outs(1)

    gsrc, gdst = _gather_k(src3, dst3, tsrc, tdst)

    be = 1024
    eff = pl.pallas_call(
        _edge_body,
        grid=(ep // be,),
        in_specs=[
            pl.BlockSpec((be, 128), lambda i: (i, 0)),
            pl.BlockSpec((be, 128), lambda i: (i, 0)),
            full((128, 128)), row(128), row(128), full((128, ef)), row(ef),
        ],
        out_specs=pl.BlockSpec((be, ef), lambda i: (i, 0)),
        out_shape=jax.ShapeDtypeStruct((ep, ef), _F32),
    )(gsrc, gdst, Wr2.astype(_BF16), br1.reshape(1, -1), br2.reshape(1, -1),
      Wrp[256:384].astype(_BF16), brp.reshape(1, -1))

    zeros_blk = jnp.zeros((zr, ef), _F32)
    thr = jnp.float32(2.0 * _MARGIN)

    @functools.partial(
        pl.kernel,
        mesh=mesh,
        out_type=jax.ShapeDtypeStruct((_NC, n_pad, ef), _F32),
        scratch_types=[
            pltpu.VMEM((_C,), _I32),
            pltpu.VMEM((_C,), _I32),
            pltpu.VMEM((2, _C), _I32),
            pltpu.VMEM((2, _C, ef), _F32),
            pltpu.VMEM((_C,), _F32),
            pltpu.VMEM((_C,), _F32),
            pltpu.VMEM((_C,), _F32),
            pltpu.VMEM((_C,), _F32),
            pltpu.VMEM_SHARED((n_pad, ef), _F32),
            pltpu.SemaphoreType.DMA,
            pltpu.SemaphoreType.DMA,
            pltpu.SemaphoreType.DMA,
            pltpu.SemaphoreType.DMA,
            pltpu.SemaphoreType.DMA,
        ],
    )
    def _scatter_k(src_hbm, dst_hbm, eff_hbm, s0_hbm, s4_hbm, z_hbm, agg_hbm,
                   idxs_v, idxd_v, idxm_v, val_v, s0s_v, s0d_v, s4s_v, s4d_v,
                   acc_sh, sem_v0, sem_v1, sem_m, sem_s0, sem_s1):
        c = lax.axis_index("c")
        s = lax.axis_index("s")
        wid = s * _NC + c
        base = wid * per_w
        sem_v = (sem_v0, sem_v1)
        sem_s = (sem_s0, sem_s1)
        pltpu.sync_copy(z_hbm, acc_sh.at[pl.ds(s * zr, zr)])
        plsc.subcore_barrier()
        pltpu.async_copy(eff_hbm.at[pl.ds(base, _C)], val_v.at[0], sem_v[0])

        def body(i2, carry):
            for b in range(2):
                i = i2 * 2 + b
                nb = 1 - b
                pltpu.sync_copy(src_hbm.at[wid, i], idxs_v)
                pltpu.sync_copy(dst_hbm.at[wid, i], idxd_v)
                g1 = pltpu.async_copy(s0_hbm.at[idxs_v], s0s_v, sem_m)
                g2 = pltpu.async_copy(s0_hbm.at[idxd_v], s0d_v, sem_m)
                g3 = pltpu.async_copy(s4_hbm.at[idxs_v], s4s_v, sem_m)
                g4 = pltpu.async_copy(s4_hbm.at[idxd_v], s4d_v, sem_m)

                @pl.when((i2 > 0) | (b > 0))
                def _():
                    pltpu.make_async_copy(
                        val_v.at[nb], acc_sh.at[idxm_v.at[nb]],
                        sem_s[nb]).wait()

                @pl.when(i + 1 < chunks)
                def _():
                    off = base + (i + 1) * _C
                    pltpu.async_copy(eff_hbm.at[pl.ds(off, _C)],
                                     val_v.at[nb], sem_v[nb])

                pltpu.make_async_copy(
                    eff_hbm.at[pl.ds(0, _C)], val_v.at[b], sem_v[b]).wait()
                g1.wait()
                g2.wait()
                g3.wait()
                g4.wait()
                for j in range(_C // _L):
                    sl = pl.ds(j * _L, _L)
                    di = idxd_v[sl]
                    relx = s0s_v[sl] - s0d_v[sl]
                    rely = s4s_v[sl] - s4d_v[sl]
                    sel = (jnp.abs(relx) > thr) | (jnp.abs(rely) > thr)
                    idxm_v[b, sl] = jnp.where(sel, di, dummy)

                pltpu.async_copy(val_v.at[b], acc_sh.at[idxm_v.at[b]],
                                 sem_s[b], add=True)
            return carry

        lax.fori_loop(0, chunks // 2, body, 0)
        pltpu.make_async_copy(
            val_v.at[1], acc_sh.at[idxm_v.at[1]], sem_s[1]).wait()
        plsc.subcore_barrier()
        pltpu.sync_copy(acc_sh.at[pl.ds(s * zr, zr)],
                        agg_hbm.at[c, pl.ds(s * zr, zr)])

    aggp = _scatter_k(src3, dst3, eff, s0, s4, zeros_blk)

    agg2 = aggp[:, :n, :]
    g_out = pl.pallas_call(
        _head_body,
        grid=(n // bn,),
        in_specs=[
            pl.BlockSpec((_NC, bn, ef), lambda i: (0, i, 0)),
            pl.BlockSpec((bn, 128), lambda i: (i, 0)),
            full((128, 128)),
            full((ef, 128)), row(128), full((128, 128)), row(128),
            full((128, g_dim)), row(g_dim),
        ],
        out_specs=pl.BlockSpec((bn, g_dim), lambda i: (i, 0)),
        out_shape=jax.ShapeDtypeStruct((n, g_dim), _F32),
    )(agg2, dvec, Wp[0:128], We1, be1.reshape(1, -1),
      We2, be2.reshape(1, -1), We3, be3.reshape(1, -1))

    return g_out


# speedup vs baseline: 3.8156x; 1.1716x over previous
import functools

import jax
import jax.numpy as jnp
from jax import lax
from jax.experimental import pallas as pl
from jax.experimental.pallas import tpu as pltpu
from jax.experimental.pallas import tpu_sc as plsc

_F32 = jnp.float32
_BF16 = jnp.bfloat16
_I32 = jnp.int32
_U32 = jnp.uint32
_MARGIN = 0.03
_NT = 4

_NC = 2
_NS = 16
_NW = _NC * _NS
_C = 128
_L = 16


def _pack_body(states_ref, ws1, bs1, ws2, bs2, wr1, wrpa, wrpb,
               wi1, bi1, wi2, bi2, wpa, wpb, bp,
               tsrc_ref, tdst_ref, d_ref):
    x = states_ref[...]
    h = jnp.maximum(x @ ws1[...] + bs1[...], 0.0)
    senc = jnp.maximum(h @ ws2[...] + bs2[...], 0.0)
    p = x @ wr1[...]
    a = senc @ wrpa[...]
    b = senc @ wrpb[...]
    hi = jnp.maximum(x @ wi1[...] + bi1[...], 0.0)
    eu = jnp.maximum(hi @ wi2[...] + bi2[...], 0.0)
    ux = jnp.abs(x[:, 0:1])
    uy = jnp.abs(x[:, _NT:_NT + 1])
    u = jnp.where((ux > 1.0 - _MARGIN) | (uy > 1.0 - _MARGIN), 1.0, 0.0)
    d_ref[...] = senc @ wpa[...] + (eu * u) @ wpb[...] + bp[...]

    def bits(v):
        return lax.bitcast_convert_type(v.astype(_BF16).astype(_F32), _U32)

    pw = bits(p) >> 16
    mask_hi = jnp.uint32(0xFFFF0000)
    tsrc_ref[...] = lax.bitcast_convert_type(pw | (bits(a) & mask_hi), _I32)
    tdst_ref[...] = lax.bitcast_convert_type(pw | (bits(b) & mask_hi), _I32)


def _edge_body(gsrc_ref, gdst_ref, wr2, br1, br2, wrpc, brp, out_ref):
    us = gsrc_ref[...]
    ud = gdst_ref[...]
    mask_hi = jnp.int32(-65536)
    ps = lax.bitcast_convert_type(us << 16, _F32)
    pd = lax.bitcast_convert_type(ud << 16, _F32)
    asrc = lax.bitcast_convert_type(us & mask_hi, _F32)
    bdst = lax.bitcast_convert_type(ud & mask_hi, _F32)
    h1 = jnp.maximum(ps - pd + br1[...], 0.0).astype(_BF16)
    er = lax.dot_general(h1, wr2[...], (((1,), (0,)), ((), ())),
                         preferred_element_type=_F32) + br2[...]
    er = jnp.maximum(er, 0.0).astype(_BF16)
    t = lax.dot_general(er, wrpc[...], (((1,), (0,)), ((), ())),
                        preferred_element_type=_F32)
    out_ref[...] = jnp.maximum(asrc + bdst + t + brp[...], 0.0)


def _head_body(agg_ref, d_ref, wp1, we1, be1, we2, be2, we3, be3, out_ref):
    agg = (agg_ref[0] + agg_ref[1]) + (agg_ref[2] + agg_ref[3])
    ne = jnp.maximum(agg @ wp1[...] + d_ref[...], 0.0)
    hh = jnp.maximum(ne @ we1[...] + be1[...], 0.0)
    hh = jnp.maximum(hh @ we2[...] + be2[...], 0.0)
    out_ref[...] = hh @ we3[...] + be3[...]


def kernel(states, edge_index, Ws1, bs1, Ws2, bs2, Wr1, br1, Wr2, br2,
           Wrp, brp, Wi1, bi1, Wi2, bi2, Wp, bp, We1, be1, We2, be2,
           We3, be3):
    n, s_dim = states.shape
    e = edge_index.shape[1]
    ef = Wrp.shape[1]
    g_dim = We3.shape[1]

    per_w = -(-e // (_NW * 4 * _C)) * 4 * _C
    ep = per_w * _NW
    chunks = per_w // _C
    n_pad = -(-n // 128) * 128
    zr = n_pad // _NS
    dummy = n_pad - 1

    src3 = jnp.pad(edge_index[0], (0, ep - e)).reshape(_NW, chunks, _C)
    dst3 = jnp.pad(edge_index[1], (0, ep - e)).reshape(_NW, chunks, _C)
    s0 = states[:, 0]
    s4 = states[:, _NT]

    bn = 2000
    full = lambda shp: pl.BlockSpec(shp, lambda i: (0,) * len(shp))
    row = lambda w: pl.BlockSpec((1, w), lambda i: (0, 0))
    tsrc, tdst, dvec = pl.pallas_call(
        _pack_body,
        grid=(n // bn,),
        in_specs=[
            pl.BlockSpec((bn, s_dim), lambda i: (i, 0)),
            full((s_dim, 128)), row(128), full((128, 128)), row(128),
            full((s_dim, 128)), full((128, 128)), full((128, 128)),
            full((s_dim, 128)), row(128), full((128, 16)), row(16),
            full((128, 128)), full((16, 128)), row(128),
        ],
        out_specs=[
            pl.BlockSpec((bn, 128), lambda i: (i, 0)),
            pl.BlockSpec((bn, 128), lambda i: (i, 0)),
            pl.BlockSpec((bn, 128), lambda i: (i, 0)),
        ],
        out_shape=[
            jax.ShapeDtypeStruct((n, 128), _I32),
            jax.ShapeDtypeStruct((n, 128), _I32),
            jax.ShapeDtypeStruct((n, 128), _F32),
        ],
    )(states, Ws1, bs1.reshape(1, -1), Ws2, bs2.reshape(1, -1),
      Wr1, Wrp[0:128], Wrp[128:256],
      Wi1, bi1.reshape(1, -1), Wi2, bi2.reshape(1, -1),
      Wp[0:128], Wp[128:144], bp.reshape(1, -1))

    mesh = plsc.VectorSubcoreMesh(core_axis_name="c", subcore_axis_name="s")
    hchunks = chunks // 2
    hper_w = per_w // 2
    hep = ep // 2

    @functools.partial(
        pl.kernel,
        mesh=mesh,
        out_type=[jax.ShapeDtypeStruct((hep, 128), _I32),
                  jax.ShapeDtypeStruct((hep, 128), _I32)],
        scratch_types=[
            pltpu.VMEM((hchunks, _C), _I32),
            pltpu.VMEM((hchunks, _C), _I32),
            pltpu.VMEM((2, _C, 128), _I32),
            pltpu.VMEM((2, _C, 128), _I32),
            pltpu.SemaphoreType.DMA,
            pltpu.SemaphoreType.DMA,
            pltpu.SemaphoreType.DMA,
            pltpu.SemaphoreType.DMA,
        ],
    )
    def _gather_k(src_hbm, dst_hbm, ts_hbm, td_hbm, gs_hbm, gd_hbm,
                  idxs_v, idxd_v, rows_s, rows_d, sem_g0, sem_g1,
                  sem_o0, sem_o1):
        c = lax.axis_index("c")
        s = lax.axis_index("s")
        wid = s * _NC + c
        base = wid * hper_w
        pltpu.sync_copy(src_hbm.at[wid], idxs_v)
        pltpu.sync_copy(dst_hbm.at[wid], idxd_v)
        sem_g = (sem_g0, sem_g1)
        sem_o = (sem_o0, sem_o1)

        def gathers(i, b):
            pltpu.async_copy(ts_hbm.at[idxs_v.at[i]], rows_s.at[b], sem_g[b])
            pltpu.async_copy(td_hbm.at[idxd_v.at[i]], rows_d.at[b], sem_g[b])

        def wait_gathers(b):
            pltpu.make_async_copy(
                ts_hbm.at[idxs_v.at[0]], rows_s.at[b], sem_g[b]).wait()
            pltpu.make_async_copy(
                td_hbm.at[idxd_v.at[0]], rows_d.at[b], sem_g[b]).wait()

        def wait_outs(b):
            pltpu.make_async_copy(
                rows_s.at[b], gs_hbm.at[pl.ds(0, _C)], sem_o[b]).wait()
            pltpu.make_async_copy(
                rows_d.at[b], gd_hbm.at[pl.ds(0, _C)], sem_o[b]).wait()

        gathers(0, 0)

        def body(i2, carry):
            for b in range(2):
                i = i2 * 2 + b
                nb = 1 - b

                if b == 0:
                    @pl.when(i2 > 0)
                    def _():
                        wait_outs(nb)
                else:
                    wait_outs(nb)

                @pl.when(i + 1 < hchunks)
                def _():
                    gathers(i + 1, nb)

                wait_gathers(b)
                off = base + i * _C
                pltpu.async_copy(rows_s.at[b], gs_hbm.at[pl.ds(off, _C)],
                                 sem_o[b])
                pltpu.async_copy(rows_d.at[b], gd_hbm.at[pl.ds(off, _C)],
                                 sem_o[b])
            return carry

        lax.fori_loop(0, hchunks // 2, body, 0)
        wait_outs(1)

    be = 1024

    def _edge_mlp(gsrc, gdst):
        return pl.pallas_call(
            _edge_body,
            grid=(hep // be,),
            in_specs=[
                pl.BlockSpec((be, 128), lambda i: (i, 0)),
                pl.BlockSpec((be, 128), lambda i: (i, 0)),
                full((128, 128)), row(128), row(128), full((128, ef)),
                row(ef),
            ],
            out_specs=pl.BlockSpec((be, ef), lambda i: (i, 0)),
            out_shape=jax.ShapeDtypeStruct((hep, ef), _F32),
        )(gsrc, gdst, Wr2.astype(_BF16), br1.reshape(1, -1),
          br2.reshape(1, -1), Wrp[256:384].astype(_BF16), brp.reshape(1, -1))

    zeros_blk = jnp.zeros((zr, ef), _F32)
    thr = jnp.float32(2.0 * _MARGIN)

    @functools.partial(
        pl.kernel,
        mesh=mesh,
        out_type=jax.ShapeDtypeStruct((_NC, n_pad, ef), _F32),
        scratch_types=[
            pltpu.VMEM((_C,), _I32),
            pltpu.VMEM((_C,), _I32),
            pltpu.VMEM((2, _C), _I32),
            pltpu.VMEM((2, _C, ef), _F32),
            pltpu.VMEM((_C,), _F32),
            pltpu.VMEM((_C,), _F32),
            pltpu.VMEM((_C,), _F32),
            pltpu.VMEM((_C,), _F32),
            pltpu.VMEM_SHARED((n_pad, ef), _F32),
            pltpu.SemaphoreType.DMA,
            pltpu.SemaphoreType.DMA,
            pltpu.SemaphoreType.DMA,
            pltpu.SemaphoreType.DMA,
            pltpu.SemaphoreType.DMA,
        ],
    )
    def _scatter_k(src_hbm, dst_hbm, eff_hbm, s0_hbm, s4_hbm, z_hbm, agg_hbm,
                   idxs_v, idxd_v, idxm_v, val_v, s0s_v, s0d_v, s4s_v, s4d_v,
                   acc_sh, sem_v0, sem_v1, sem_m, sem_s0, sem_s1):
        c = lax.axis_index("c")
        s = lax.axis_index("s")
        wid = s * _NC + c
        base = wid * hper_w
        sem_v = (sem_v0, sem_v1)
        sem_s = (sem_s0, sem_s1)
        pltpu.sync_copy(z_hbm, acc_sh.at[pl.ds(s * zr, zr)])
        plsc.subcore_barrier()
        pltpu.async_copy(eff_hbm.at[pl.ds(base, _C)], val_v.at[0], sem_v[0])

        def body(i2, carry):
            for b in range(2):
                i = i2 * 2 + b
                nb = 1 - b
                pltpu.sync_copy(src_hbm.at[wid, i], idxs_v)
                pltpu.sync_copy(dst_hbm.at[wid, i], idxd_v)
                g1 = pltpu.async_copy(s0_hbm.at[idxs_v], s0s_v, sem_m)
                g2 = pltpu.async_copy(s0_hbm.at[idxd_v], s0d_v, sem_m)
                g3 = pltpu.async_copy(s4_hbm.at[idxs_v], s4s_v, sem_m)
                g4 = pltpu.async_copy(s4_hbm.at[idxd_v], s4d_v, sem_m)

                @pl.when((i2 > 0) | (b > 0))
                def _():
                    pltpu.make_async_copy(
                        val_v.at[nb], acc_sh.at[idxm_v.at[nb]],
                        sem_s[nb]).wait()

                @pl.when(i + 1 < hchunks)
                def _():
                    off = base + (i + 1) * _C
                    pltpu.async_copy(eff_hbm.at[pl.ds(off, _C)],
                                     val_v.at[nb], sem_v[nb])

                pltpu.make_async_copy(
                    eff_hbm.at[pl.ds(0, _C)], val_v.at[b], sem_v[b]).wait()
                g1.wait()
                g2.wait()
                g3.wait()
                g4.wait()
                for j in range(_C // _L):
                    sl = pl.ds(j * _L, _L)
                    di = idxd_v[sl]
                    relx = s0s_v[sl] - s0d_v[sl]
                    rely = s4s_v[sl] - s4d_v[sl]
                    sel = (jnp.abs(relx) > thr) | (jnp.abs(rely) > thr)
                    idxm_v[b, sl] = jnp.where(sel, di, dummy)

                pltpu.async_copy(val_v.at[b], acc_sh.at[idxm_v.at[b]],
                                 sem_s[b], add=True)
            return carry

        lax.fori_loop(0, hchunks // 2, body, 0)
        pltpu.make_async_copy(
            val_v.at[1], acc_sh.at[idxm_v.at[1]], sem_s[1]).wait()
        plsc.subcore_barrier()
        pltpu.sync_copy(acc_sh.at[pl.ds(s * zr, zr)],
                        agg_hbm.at[c, pl.ds(s * zr, zr)])

    parts = []
    for h in range(2):
        sl = slice(h * hchunks, (h + 1) * hchunks)
        src_h = src3[:, sl]
        dst_h = dst3[:, sl]
        gsrc, gdst = _gather_k(src_h, dst_h, tsrc, tdst)
        eff_h = _edge_mlp(gsrc, gdst)
        parts.append(_scatter_k(src_h, dst_h, eff_h, s0, s4, zeros_blk))
    aggp = jnp.concatenate(parts, axis=0)

    agg2 = aggp[:, :n, :]
    g_out = pl.pallas_call(
        _head_body,
        grid=(n // bn,),
        in_specs=[
            pl.BlockSpec((2 * _NC, bn, ef), lambda i: (0, i, 0)),
            pl.BlockSpec((bn, 128), lambda i: (i, 0)),
            full((128, 128)),
            full((ef, 128)), row(128), full((128, 128)), row(128),
            full((128, g_dim)), row(g_dim),
        ],
        out_specs=pl.BlockSpec((bn, g_dim), lambda i: (i, 0)),
        out_shape=jax.ShapeDtypeStruct((n, g_dim), _F32),
    )(agg2, dvec, Wp[0:128], We1, be1.reshape(1, -1),
      We2, be2.reshape(1, -1), We3, be3.reshape(1, -1))

    return g_out
